# Initial kernel scaffold; baseline (speedup 1.0000x reference)
import jax
import jax.numpy as jnp
from jax.experimental import pallas as pl


def kernel(x, fc_table, emb_table, lin_W, lin_b, W1, b1, W2, b2, W3, b3):
    raise NotImplementedError("write your pallas kernel here")


# speedup vs baseline: 1.0740x; 1.0740x over previous
import jax
import jax.numpy as jnp
from jax import lax
from jax.experimental import pallas as pl
from jax.experimental.pallas import tpu as pltpu
from jax.experimental.pallas import tpu_sc as plsc

B = 16384
F = 26
K = 16
EMB = F * K
NC, NS = 2, 16
NW = NC * NS
TOTAL = B * F
PER_W = TOTAL // NW
CHUNK = 128
NCHUNK = PER_W // CHUNK


def _sc_gather_body(x_hbm, emb_hbm, fc_hbm, rows_out, fcpart_out,
                    idx_v, rowbuf, fcbuf, acc_v, gsem, fsem):
    wid = lax.axis_index("s") * NC + lax.axis_index("c")
    base_chunk = wid * NCHUNK

    pltpu.sync_copy(x_hbm.at[pl.ds(base_chunk, NCHUNK)], idx_v)

    acc_v[...] = jnp.zeros((16,), jnp.float32)

    def step(j, _):
        idx_row = idx_v.at[j]
        pltpu.async_copy(emb_hbm.at[idx_row], rowbuf, gsem).wait()
        pltpu.sync_copy(rowbuf, rows_out.at[pl.ds((base_chunk + j) * CHUNK, CHUNK)])
        pltpu.async_copy(fc_hbm.at[idx_row], fcbuf, fsem).wait()
        acc = acc_v[...]
        for i in range(CHUNK // 16):
            acc = acc + fcbuf[pl.ds(i * 16, 16)]
        acc_v[...] = acc
        return 0

    lax.fori_loop(0, NCHUNK, step, 0)
    pltpu.sync_copy(acc_v, fcpart_out.at[wid])


def _sc_gather(x_flat2d, emb_table, fc_flat):
    mesh = plsc.VectorSubcoreMesh(
        core_axis_name="c", subcore_axis_name="s",
        num_cores=NC, num_subcores=NS)
    return pl.kernel(
        _sc_gather_body,
        out_type=[
            jax.ShapeDtypeStruct((TOTAL, K), jnp.float32),
            jax.ShapeDtypeStruct((NW, 16), jnp.float32),
        ],
        mesh=mesh,
        scratch_types=[
            pltpu.VMEM((NCHUNK, CHUNK), jnp.int32),
            pltpu.VMEM((CHUNK, K), jnp.float32),
            pltpu.VMEM((CHUNK,), jnp.float32),
            pltpu.VMEM((16,), jnp.float32),
            pltpu.SemaphoreType.DMA,
            pltpu.SemaphoreType.DMA,
        ],
        compiler_params=pltpu.CompilerParams(use_tc_tiling_on_sc=False),
    )(x_flat2d, emb_table, fc_flat)


BLK = 2048


def _dense_body(h_ref, part_ref, linw_ref, linb_ref, w1_ref, b1_ref,
                w2_ref, b2_ref, w3_ref, b3_ref, y_ref):
    h = h_ref[...]
    r = lax.broadcasted_iota(jnp.int32, (EMB, K), 0)
    c = lax.broadcasted_iota(jnp.int32, (EMB, K), 1)
    sel = jnp.where((r % K) == c, 1.0, 0.0)
    s = jnp.dot(h, sel, preferred_element_type=jnp.float32)
    sq = jnp.dot(h * h, sel, preferred_element_type=jnp.float32)
    inter = jnp.sum(s * s - sq, axis=1, keepdims=True)

    tot = jnp.sum(part_ref[...])
    linw = jnp.sum(linw_ref[...])
    linb = jnp.sum(linb_ref[...])
    linear_term = linw * tot + jnp.float32(TOTAL) * linb

    h1 = jnp.maximum(
        jnp.dot(h, w1_ref[...], preferred_element_type=jnp.float32)
        + b1_ref[...], 0.0)
    h2 = jnp.maximum(
        jnp.dot(h1, w2_ref[...], preferred_element_type=jnp.float32)
        + b2_ref[...], 0.0)
    m = jnp.dot(h2, w3_ref[...], preferred_element_type=jnp.float32) + b3_ref[...]

    z = linear_term + 0.5 * inter + m
    y_ref[...] = 1.0 / (1.0 + jnp.exp(-z))


def _dense(hmat, fcpart, lin_W, lin_b, W1, b1, W2, b2, W3, b3):
    grid = (B // BLK,)
    zero = lambda i: (0, 0)
    return pl.pallas_call(
        _dense_body,
        out_shape=jax.ShapeDtypeStruct((B, 1), jnp.float32),
        grid=grid,
        in_specs=[
            pl.BlockSpec((BLK, EMB), lambda i: (i, 0)),
            pl.BlockSpec((NW, 16), zero),
            pl.BlockSpec((1, 1), zero),
            pl.BlockSpec((1, 1), zero),
            pl.BlockSpec((EMB, 128), zero),
            pl.BlockSpec((1, 128), zero),
            pl.BlockSpec((128, 64), zero),
            pl.BlockSpec((1, 64), zero),
            pl.BlockSpec((64, 1), zero),
            pl.BlockSpec((1, 1), zero),
        ],
        out_specs=pl.BlockSpec((BLK, 1), lambda i: (i, 0)),
    )(hmat, fcpart, lin_W, lin_b, W1, b1, W2, b2, W3, b3)


def kernel(x, fc_table, emb_table, lin_W, lin_b, W1, b1, W2, b2, W3, b3):
    x_flat2d = x.reshape(TOTAL // CHUNK, CHUNK)
    fc_flat = fc_table.reshape(-1)
    rows, fcpart = _sc_gather(x_flat2d, emb_table, fc_flat)
    hmat = rows.reshape(B, EMB)
    return _dense(hmat, fcpart, lin_W, lin_b.reshape(1, 1),
                  W1, b1.reshape(1, -1), W2, b2.reshape(1, -1),
                  W3, b3.reshape(1, 1))


# speedup vs baseline: 1.3623x; 1.2684x over previous
import jax
import jax.numpy as jnp
from jax import lax
from jax.experimental import pallas as pl
from jax.experimental.pallas import tpu as pltpu
from jax.experimental.pallas import tpu_sc as plsc

B = 16384
F = 26
K = 16
EMB = F * K
NC, NS = 2, 16
NW = NC * NS
TOTAL = B * F
PER_W = TOTAL // NW
CHUNK = 128
NCHUNK = PER_W // CHUNK
KB = 13
NGRP = NCHUNK // KB
GROWS = KB * CHUNK


def _sc_gather_body(x_hbm, emb_hbm, fc_hbm, rows_out, fc_out,
                    idx_v, rows2, fcall, esem0, esem1, fsem, osem0, osem1):
    wid = lax.axis_index("s") * NC + lax.axis_index("c")
    base_chunk = wid * NCHUNK
    base_row = wid * PER_W

    pltpu.sync_copy(x_hbm.at[pl.ds(base_chunk, NCHUNK)], idx_v)

    esem = (esem0, esem1)
    osem = (osem0, osem1)
    edesc = [None] * NGRP
    odesc = [None] * NGRP

    def fire_group(g):
        slot = g & 1
        ds_ = []
        for j in range(KB):
            idx_row = idx_v.at[g * KB + j]
            ds_.append(pltpu.async_copy(
                emb_hbm.at[idx_row],
                rows2.at[slot].at[pl.ds(j * CHUNK, CHUNK)],
                esem[slot]))
            pltpu.async_copy(fc_hbm.at[idx_row], fcall.at[g * KB + j], fsem)
        edesc[g] = ds_

    def drain_and_writeback(g):
        slot = g & 1
        for d in edesc[g]:
            d.wait()
        odesc[g] = pltpu.async_copy(
            rows2.at[slot],
            rows_out.at[pl.ds(base_row + g * GROWS, GROWS)],
            osem[slot])

    for g in range(NGRP):
        if g >= 2:
            odesc[g - 2].wait()
        fire_group(g)
        if g >= 1:
            drain_and_writeback(g - 1)
    drain_and_writeback(NGRP - 1)
    odesc[NGRP - 2].wait()
    odesc[NGRP - 1].wait()

    pltpu.make_async_copy(fc_out.at[pl.ds(0, NCHUNK)], fcall, fsem).wait()
    pltpu.sync_copy(fcall, fc_out.at[pl.ds(base_chunk, NCHUNK)])


def _sc_gather(x_flat2d, emb_table, fc_flat):
    mesh = plsc.VectorSubcoreMesh(
        core_axis_name="c", subcore_axis_name="s",
        num_cores=NC, num_subcores=NS)
    return pl.kernel(
        _sc_gather_body,
        out_type=[
            jax.ShapeDtypeStruct((TOTAL, K), jnp.float32),
            jax.ShapeDtypeStruct((TOTAL // CHUNK, CHUNK), jnp.float32),
        ],
        mesh=mesh,
        scratch_types=[
            pltpu.VMEM((NCHUNK, CHUNK), jnp.int32),
            pltpu.VMEM((2, GROWS, K), jnp.float32),
            pltpu.VMEM((NCHUNK, CHUNK), jnp.float32),
            pltpu.SemaphoreType.DMA,
            pltpu.SemaphoreType.DMA,
            pltpu.SemaphoreType.DMA,
            pltpu.SemaphoreType.DMA,
            pltpu.SemaphoreType.DMA,
        ],
        compiler_params=pltpu.CompilerParams(use_tc_tiling_on_sc=False),
    )(x_flat2d, emb_table, fc_flat)


BLK = 2048


def _dense_body(h_ref, fc_ref, linw_ref, linb_ref, w1_ref, b1_ref,
                w2_ref, b2_ref, w3_ref, b3_ref, y_ref):
    h = h_ref[...]
    r = lax.broadcasted_iota(jnp.int32, (EMB, K), 0)
    c = lax.broadcasted_iota(jnp.int32, (EMB, K), 1)
    sel = jnp.where((r % K) == c, 1.0, 0.0)
    s = jnp.dot(h, sel, preferred_element_type=jnp.float32)
    sq = jnp.dot(h * h, sel, preferred_element_type=jnp.float32)
    inter = jnp.sum(s * s - sq, axis=1, keepdims=True)

    tot = jnp.sum(fc_ref[...])
    linw = jnp.sum(linw_ref[...])
    linb = jnp.sum(linb_ref[...])
    linear_term = linw * tot + jnp.float32(TOTAL) * linb

    h1 = jnp.maximum(
        jnp.dot(h, w1_ref[...], preferred_element_type=jnp.float32)
        + b1_ref[...], 0.0)
    h2 = jnp.maximum(
        jnp.dot(h1, w2_ref[...], preferred_element_type=jnp.float32)
        + b2_ref[...], 0.0)
    m = jnp.dot(h2, w3_ref[...], preferred_element_type=jnp.float32) + b3_ref[...]

    z = linear_term + 0.5 * inter + m
    y_ref[...] = 1.0 / (1.0 + jnp.exp(-z))


def _dense(hmat, fcvals, lin_W, lin_b, W1, b1, W2, b2, W3, b3):
    grid = (B // BLK,)
    zero = lambda i: (0, 0)
    return pl.pallas_call(
        _dense_body,
        out_shape=jax.ShapeDtypeStruct((B, 1), jnp.float32),
        grid=grid,
        in_specs=[
            pl.BlockSpec((BLK, EMB), lambda i: (i, 0)),
            pl.BlockSpec((TOTAL // CHUNK, CHUNK), zero),
            pl.BlockSpec((1, 1), zero),
            pl.BlockSpec((1, 1), zero),
            pl.BlockSpec((EMB, 128), zero),
            pl.BlockSpec((1, 128), zero),
            pl.BlockSpec((128, 64), zero),
            pl.BlockSpec((1, 64), zero),
            pl.BlockSpec((64, 1), zero),
            pl.BlockSpec((1, 1), zero),
        ],
        out_specs=pl.BlockSpec((BLK, 1), lambda i: (i, 0)),
    )(hmat, fcvals, lin_W, lin_b, W1, b1, W2, b2, W3, b3)


def kernel(x, fc_table, emb_table, lin_W, lin_b, W1, b1, W2, b2, W3, b3):
    x_flat2d = x.reshape(TOTAL // CHUNK, CHUNK)
    fc_flat = fc_table.reshape(-1)
    rows, fcvals = _sc_gather(x_flat2d, emb_table, fc_flat)
    hmat = rows.reshape(B, EMB)
    return _dense(hmat, fcvals, lin_W, lin_b.reshape(1, 1),
                  W1, b1.reshape(1, -1), W2, b2.reshape(1, -1),
                  W3, b3.reshape(1, 1))
